# Initial kernel scaffold; baseline (speedup 1.0000x reference)
import jax
import jax.numpy as jnp
from jax.experimental import pallas as pl


def kernel(x, e, W1, as1, ad1, b1, W2, as2, ad2, b2):
    raise NotImplementedError("write your pallas kernel here")


# speedup vs baseline: 14.7626x; 14.7626x over previous
import functools

import jax
import jax.numpy as jnp
from jax import lax
from jax.experimental import pallas as pl
from jax.experimental.pallas import tpu as pltpu
from jax.experimental.pallas import tpu_sc as plsc

N = 10000
E = 320000
D = 128

NC = 2
NS = 16
NW = NC * NS

CHUNK = 128
G8 = 8
NP = 10240
NPT = NP // NS
EPG = 352
EP = EPG * G8 * CHUNK
TG = EPG // NS
HG = TG // 2
WG = EPG // NW

_f32 = jnp.float32
_i32 = jnp.int32


BM = 512


def _mm1_body(x_ref, w_ref, a2_ref, h_ref, as_ref, ad_ref):
    hb = jnp.dot(x_ref[...], w_ref[...], preferred_element_type=_f32)
    h_ref[...] = hb
    av = lax.dot_general(a2_ref[...], hb, (((1,), (1,)), ((), ())))
    as_ref[...] = av[0:1]
    ad_ref[...] = av[1:2]


def _mm2_body(o0_ref, o1_ref, b_ref, w_ref, a2_ref, h_ref, as_ref, ad_ref):
    yb = o0_ref[...] + o1_ref[...] + b_ref[...]
    hb = jnp.dot(yb, w_ref[...], preferred_element_type=_f32)
    h_ref[...] = hb
    av = lax.dot_general(a2_ref[...], hb, (((1,), (1,)), ((), ())))
    as_ref[...] = av[0:1]
    ad_ref[...] = av[1:2]


def _final_body(o0_ref, o1_ref, b_ref, y_ref):
    y_ref[...] = jnp.maximum(o0_ref[...] + o1_ref[...] + b_ref[...], 0.0)


_row_spec = pl.BlockSpec((BM, D), lambda i: (i, 0))
_w_spec = pl.BlockSpec((D, D), lambda i: (0, 0))
_a2_spec = pl.BlockSpec((2, D), lambda i: (0, 0))
_b_spec = pl.BlockSpec((1, D), lambda i: (0, 0))
_av_spec = pl.BlockSpec((1, BM), lambda i: (0, i))

_mm_out = [
    jax.ShapeDtypeStruct((NP, D), _f32),
    jax.ShapeDtypeStruct((1, NP), _f32),
    jax.ShapeDtypeStruct((1, NP), _f32),
]

_mm1 = pl.pallas_call(
    _mm1_body,
    grid=(NP // BM,),
    in_specs=[_row_spec, _w_spec, _a2_spec],
    out_specs=[_row_spec, _av_spec, _av_spec],
    out_shape=_mm_out,
)

_mm2 = pl.pallas_call(
    _mm2_body,
    grid=(NP // BM,),
    in_specs=[_row_spec, _row_spec, _b_spec, _w_spec, _a2_spec],
    out_specs=[_row_spec, _av_spec, _av_spec],
    out_shape=_mm_out,
)

_final = pl.pallas_call(
    _final_body,
    grid=(NP // BM,),
    in_specs=[_row_spec, _row_spec, _b_spec],
    out_specs=_row_spec,
    out_shape=jax.ShapeDtypeStruct((NP, D), _f32),
)


_mesh = plsc.VectorSubcoreMesh(core_axis_name="c", subcore_axis_name="s")
_sc_params = pltpu.CompilerParams(use_tc_tiling_on_sc=False)


@functools.partial(
    pl.kernel,
    out_type=[
        jax.ShapeDtypeStruct((EPG, G8, CHUNK), _f32),
        jax.ShapeDtypeStruct((NP,), _f32),
    ],
    mesh=_mesh,
    scratch_types=[
        pltpu.VMEM((TG, G8, CHUNK), _i32),
        pltpu.VMEM((TG, G8, CHUNK), _i32),
        pltpu.VMEM((CHUNK,), _f32),
        pltpu.VMEM((CHUNK,), _f32),
        pltpu.VMEM((G8, CHUNK), _f32),
        pltpu.VMEM((CHUNK,), _i32),
        pltpu.VMEM((NPT,), _f32),
        pltpu.VMEM_SHARED((NP,), _f32),
    ],
    compiler_params=_sc_params,
)
def _sc_b1(asv, adv, srcp, dstp, wv, rden,
           srcl, dstl, asg, adg, wb, dsti, db, dsh):
    c = lax.axis_index("c")
    t = lax.axis_index("s")

    def _zero(i, _):
        db[pl.ds(i * 16, 16)] = jnp.zeros((16,), _f32)
        return 0
    lax.fori_loop(0, NPT // 16, _zero, 0)
    pltpu.sync_copy(db, dsh.at[pl.ds(t * NPT, NPT)])
    plsc.subcore_barrier()

    pltpu.sync_copy(srcp.at[pl.ds(t * TG, TG)], srcl)
    pltpu.sync_copy(dstp.at[pl.ds(t * TG, TG)], dstl)

    def _group(jo, _):
        for ks in range(G8):
            pltpu.sync_copy(asv.at[srcl.at[jo].at[ks]], asg)
            for k in range(CHUNK // 16):
                dsti[pl.ds(k * 16, 16)] = dstl[jo, ks, pl.ds(k * 16, 16)]
            pltpu.sync_copy(adv.at[dsti], adg)
            for k in range(CHUNK // 16):
                v = asg[pl.ds(k * 16, 16)] + adg[pl.ds(k * 16, 16)]
                v = jnp.where(v >= 0.0, v, 0.2 * v)
                wb[ks, pl.ds(k * 16, 16)] = jnp.exp(v)
            pltpu.sync_copy(wb.at[ks], dsh.at[dsti], add=True)

        @pl.when((jo < HG) == (c == 0))
        def _():
            pltpu.sync_copy(wb, wv.at[t * TG + jo])
        return 0
    lax.fori_loop(0, TG, _group, 0)
    plsc.subcore_barrier()

    pltpu.sync_copy(dsh.at[pl.ds(t * NPT, NPT)], db)

    def _recip(i, _):
        db[pl.ds(i * 16, 16)] = 1.0 / db[pl.ds(i * 16, 16)]
        return 0
    lax.fori_loop(0, NPT // 16, _recip, 0)

    @pl.when(c == 0)
    def _():
        pltpu.sync_copy(db, rden.at[pl.ds(t * NPT, NPT)])


@functools.partial(
    pl.kernel,
    out_type=[
        jax.ShapeDtypeStruct((NP, D), _f32),
        jax.ShapeDtypeStruct((NP, D), _f32),
    ],
    mesh=_mesh,
    scratch_types=[
        pltpu.VMEM((G8, CHUNK), _i32),
        pltpu.VMEM((G8, CHUNK), _i32),
        pltpu.VMEM((G8, CHUNK), _f32),
        pltpu.VMEM((CHUNK,), _i32),
        pltpu.VMEM((CHUNK,), _f32),
        pltpu.VMEM((CHUNK,), _f32),
        pltpu.VMEM((CHUNK, D), _f32),
        pltpu.VMEM_SHARED((NP, D), _f32),
    ],
    compiler_params=_sc_params,
)
def _sc_b2(h, wv, rden, srcp, dstp, o0, o1,
           srcg, dstg, wg, dsti, rd, attv, hrows, osh):
    c = lax.axis_index("c")
    t = lax.axis_index("s")
    wid = c * NS + t

    def _zrow(i, _):
        for k in range(D // 16):
            hrows[i, pl.ds(k * 16, 16)] = jnp.zeros((16,), _f32)
        return 0
    lax.fori_loop(0, CHUNK, _zrow, 0)

    def _zcp(r, _):
        pltpu.sync_copy(hrows, osh.at[pl.ds(t * NPT + r * CHUNK, CHUNK)])
        return 0
    lax.fori_loop(0, NPT // CHUNK, _zcp, 0)
    plsc.subcore_barrier()

    def _group(jo, _):
        pltpu.sync_copy(srcp.at[wid * WG + jo], srcg)
        pltpu.sync_copy(dstp.at[wid * WG + jo], dstg)
        pltpu.sync_copy(wv.at[wid * WG + jo], wg)
        for ks in range(G8):
            pltpu.sync_copy(h.at[srcg.at[ks]], hrows)
            for k in range(CHUNK // 16):
                dsti[pl.ds(k * 16, 16)] = dstg[ks, pl.ds(k * 16, 16)]
            pltpu.sync_copy(rden.at[dsti], rd)
            for k in range(CHUNK // 16):
                attv[pl.ds(k * 16, 16)] = (
                    wg[ks, pl.ds(k * 16, 16)] * rd[pl.ds(k * 16, 16)]
                )

            def _rgroup(r, _):
                av = attv[pl.ds(r * 16, 16)]
                base = r * 16
                for i16 in range(16):
                    a = jnp.zeros((16,), _f32) + av[i16]
                    for k in range(D // 16):
                        hrows[base + i16, pl.ds(k * 16, 16)] = (
                            hrows[base + i16, pl.ds(k * 16, 16)] * a
                        )
                return 0
            lax.fori_loop(0, CHUNK // 16, _rgroup, 0)

            pltpu.sync_copy(hrows, osh.at[dsti], add=True)
        return 0
    lax.fori_loop(0, WG, _group, 0)
    plsc.subcore_barrier()

    @pl.when(c == 0)
    def _():
        def _cp(r, _):
            sl = pl.ds(t * NPT + r * CHUNK, CHUNK)
            pltpu.sync_copy(osh.at[sl], o0.at[sl])
            return 0
        lax.fori_loop(0, NPT // CHUNK, _cp, 0)

    @pl.when(c == 1)
    def _():
        def _cp(r, _):
            sl = pl.ds(t * NPT + r * CHUNK, CHUNK)
            pltpu.sync_copy(osh.at[sl], o1.at[sl])
            return 0
        lax.fori_loop(0, NPT // CHUNK, _cp, 0)


def kernel(x, e, W1, as1, ad1, b1, W2, as2, ad2, b2):
    xp = jnp.zeros((NP, D), _f32).at[:N].set(x)
    loop = jnp.arange(N, dtype=_i32)
    npad = EP - E - N
    pad = N + jnp.arange(npad, dtype=_i32) % (NP - N)
    srcp = jnp.concatenate([e[0], loop, pad]).reshape(EPG, G8, CHUNK)
    dstp = jnp.concatenate([e[1], loop, pad]).reshape(EPG, G8, CHUNK)
    a21 = jnp.stack([as1, ad1])
    a22 = jnp.stack([as2, ad2])
    b1r = b1.reshape(1, D)
    b2r = b2.reshape(1, D)

    h1, asv1, adv1 = _mm1(xp, W1, a21)
    w1, rd1 = _sc_b1(asv1.reshape(NP), adv1.reshape(NP), srcp, dstp)
    o1a, o1b = _sc_b2(h1, w1, rd1, srcp, dstp)

    h2, asv2, adv2 = _mm2(o1a, o1b, b1r, W2, a22)
    w2, rd2 = _sc_b1(asv2.reshape(NP), adv2.reshape(NP), srcp, dstp)
    o2a, o2b = _sc_b2(h2, w2, rd2, srcp, dstp)

    y = _final(o2a, o2b, b2r)
    return y[:N]


# speedup vs baseline: 26.9806x; 1.8276x over previous
import functools

import jax
import jax.numpy as jnp
from jax import lax
from jax.experimental import pallas as pl
from jax.experimental.pallas import tpu as pltpu
from jax.experimental.pallas import tpu_sc as plsc

N = 10000
E = 320000
D = 128

NC = 2
NS = 16
NW = NC * NS

CHUNK = 128
G8 = 8
NP = 10240
NPT = NP // NS
EPG = 352
EP = EPG * G8 * CHUNK
TG = EPG // NS
HG = TG // 2
WG = EPG // NW

_f32 = jnp.float32
_i32 = jnp.int32


BM = 512


def _mm1_body(x_ref, w_ref, a2_ref, h_ref, as_ref, ad_ref):
    hb = jnp.dot(x_ref[...], w_ref[...], preferred_element_type=_f32)
    h_ref[...] = hb
    av = lax.dot_general(a2_ref[...], hb, (((1,), (1,)), ((), ())))
    as_ref[...] = av[0:1]
    ad_ref[...] = av[1:2]


def _mm2_body(o0_ref, o1_ref, b_ref, w_ref, a2_ref, h_ref, as_ref, ad_ref):
    yb = o0_ref[...] + o1_ref[...] + b_ref[...]
    hb = jnp.dot(yb, w_ref[...], preferred_element_type=_f32)
    h_ref[...] = hb
    av = lax.dot_general(a2_ref[...], hb, (((1,), (1,)), ((), ())))
    as_ref[...] = av[0:1]
    ad_ref[...] = av[1:2]


def _final_body(o0_ref, o1_ref, b_ref, y_ref):
    y_ref[...] = jnp.maximum(o0_ref[...] + o1_ref[...] + b_ref[...], 0.0)


_row_spec = pl.BlockSpec((BM, D), lambda i: (i, 0))
_w_spec = pl.BlockSpec((D, D), lambda i: (0, 0))
_a2_spec = pl.BlockSpec((2, D), lambda i: (0, 0))
_b_spec = pl.BlockSpec((1, D), lambda i: (0, 0))
_av_spec = pl.BlockSpec((1, BM), lambda i: (0, i))

_mm_out = [
    jax.ShapeDtypeStruct((NP, D), _f32),
    jax.ShapeDtypeStruct((1, NP), _f32),
    jax.ShapeDtypeStruct((1, NP), _f32),
]

_mm1 = pl.pallas_call(
    _mm1_body,
    grid=(NP // BM,),
    in_specs=[_row_spec, _w_spec, _a2_spec],
    out_specs=[_row_spec, _av_spec, _av_spec],
    out_shape=_mm_out,
)

_mm2 = pl.pallas_call(
    _mm2_body,
    grid=(NP // BM,),
    in_specs=[_row_spec, _row_spec, _b_spec, _w_spec, _a2_spec],
    out_specs=[_row_spec, _av_spec, _av_spec],
    out_shape=_mm_out,
)

_final = pl.pallas_call(
    _final_body,
    grid=(NP // BM,),
    in_specs=[_row_spec, _row_spec, _b_spec],
    out_specs=_row_spec,
    out_shape=jax.ShapeDtypeStruct((NP, D), _f32),
)


_mesh = plsc.VectorSubcoreMesh(core_axis_name="c", subcore_axis_name="s")
_sc_params = pltpu.CompilerParams(use_tc_tiling_on_sc=False)


@functools.partial(
    pl.kernel,
    out_type=jax.ShapeDtypeStruct((EPG, G8, CHUNK), _f32),
    mesh=_mesh,
    scratch_types=[
        pltpu.VMEM((TG, G8, CHUNK), _i32),
        pltpu.VMEM((TG, G8, CHUNK), _i32),
        pltpu.VMEM((TG, G8, CHUNK), _f32),
        pltpu.VMEM((CHUNK,), _f32),
        pltpu.VMEM((CHUNK,), _f32),
        pltpu.VMEM((CHUNK,), _f32),
        pltpu.VMEM((CHUNK,), _f32),
        pltpu.VMEM((CHUNK,), _i32),
        pltpu.VMEM((CHUNK,), _f32),
        pltpu.VMEM((G8, CHUNK), _f32),
        pltpu.VMEM((NPT,), _f32),
        pltpu.VMEM_SHARED((NP,), _f32),
        pltpu.SemaphoreType.DMA,
        pltpu.SemaphoreType.DMA,
    ],
    compiler_params=_sc_params,
)
def _sc_b1(asv, adv, srcp, dstp, attv,
           srcl, dstl, wfull, asg0, asg1, adg0, adg1, dsti, rd, attb, db,
           dsh, sem0, sem1):
    c = lax.axis_index("c")
    t = lax.axis_index("s")
    asg = (asg0, asg1)
    adg = (adg0, adg1)
    sem = (sem0, sem1)

    def _zero(i, _):
        db[pl.ds(i * 16, 16)] = jnp.zeros((16,), _f32)
        return 0
    lax.fori_loop(0, NPT // 16, _zero, 0)
    pltpu.sync_copy(db, dsh.at[pl.ds(t * NPT, NPT)])
    plsc.subcore_barrier()

    pltpu.sync_copy(srcp.at[pl.ds(t * TG, TG)], srcl)
    pltpu.sync_copy(dstp.at[pl.ds(t * TG, TG)], dstl)

    for b in range(2):
        pltpu.async_copy(asv.at[srcl.at[0].at[b]], asg[b], sem[b])
        pltpu.async_copy(adv.at[dstl.at[0].at[b]], adg[b], sem[b])

    def _group(jo, _):
        for ks in range(G8):
            b = ks % 2
            pltpu.make_async_copy(
                asv.at[srcl.at[jo].at[ks]], asg[b], sem[b]).wait()
            pltpu.make_async_copy(
                adv.at[dstl.at[jo].at[ks]], adg[b], sem[b]).wait()
            for k in range(CHUNK // 16):
                dsti[pl.ds(k * 16, 16)] = dstl[jo, ks, pl.ds(k * 16, 16)]
            for k in range(CHUNK // 16):
                v = (asg[b][pl.ds(k * 16, 16)]
                     + adg[b][pl.ds(k * 16, 16)])
                v = jnp.where(v >= 0.0, v, 0.2 * v)
                wfull[jo, ks, pl.ds(k * 16, 16)] = jnp.exp(v)
            pltpu.sync_copy(wfull.at[jo].at[ks], dsh.at[dsti], add=True)
            if ks < G8 - 2:
                pltpu.async_copy(
                    asv.at[srcl.at[jo].at[ks + 2]], asg[b], sem[b])
                pltpu.async_copy(
                    adv.at[dstl.at[jo].at[ks + 2]], adg[b], sem[b])
            else:
                @pl.when(jo + 1 < TG)
                def _():
                    pltpu.async_copy(
                        asv.at[srcl.at[jo + 1].at[ks - 6]], asg[b], sem[b])
                    pltpu.async_copy(
                        adv.at[dstl.at[jo + 1].at[ks - 6]], adg[b], sem[b])
        return 0
    lax.fori_loop(0, TG, _group, 0)
    plsc.subcore_barrier()

    pltpu.sync_copy(dsh.at[pl.ds(t * NPT, NPT)], db)

    def _recip(i, _):
        db[pl.ds(i * 16, 16)] = 1.0 / db[pl.ds(i * 16, 16)]
        return 0
    lax.fori_loop(0, NPT // 16, _recip, 0)
    pltpu.sync_copy(db, dsh.at[pl.ds(t * NPT, NPT)])
    plsc.subcore_barrier()

    def _att(jo2, _):
        gsel = jo2 + c * HG
        for ks in range(G8):
            for k in range(CHUNK // 16):
                dsti[pl.ds(k * 16, 16)] = dstl[gsel, ks, pl.ds(k * 16, 16)]
            pltpu.sync_copy(dsh.at[dsti], rd)
            for k in range(CHUNK // 16):
                attb[ks, pl.ds(k * 16, 16)] = (
                    wfull[gsel, ks, pl.ds(k * 16, 16)]
                    * rd[pl.ds(k * 16, 16)]
                )
        pltpu.sync_copy(attb, attv.at[t * TG + gsel])
        return 0
    lax.fori_loop(0, HG, _att, 0)


@functools.partial(
    pl.kernel,
    out_type=[
        jax.ShapeDtypeStruct((NP, D), _f32),
        jax.ShapeDtypeStruct((NP, D), _f32),
    ],
    mesh=_mesh,
    scratch_types=[
        pltpu.VMEM((WG, G8, CHUNK), _i32),
        pltpu.VMEM((G8, CHUNK), _i32),
        pltpu.VMEM((G8, CHUNK), _f32),
        pltpu.VMEM((CHUNK,), _i32),
        pltpu.VMEM((CHUNK, D), _f32),
        pltpu.VMEM((CHUNK, D), _f32),
        pltpu.VMEM_SHARED((NP, D), _f32),
        pltpu.SemaphoreType.DMA,
        pltpu.SemaphoreType.DMA,
    ],
    compiler_params=_sc_params,
)
def _sc_b2(h, attv, srcp, dstp, o0, o1,
           srcl, dstg, attg, dsti, hrows0, hrows1, osh, sem0, sem1):
    c = lax.axis_index("c")
    t = lax.axis_index("s")
    wid = c * NS + t
    base = t * TG + c * HG
    hrows = (hrows0, hrows1)
    sem = (sem0, sem1)

    def _zrow(i, _):
        for k in range(D // 16):
            hrows0[i, pl.ds(k * 16, 16)] = jnp.zeros((16,), _f32)
        return 0
    lax.fori_loop(0, CHUNK, _zrow, 0)

    def _zcp(r, _):
        pltpu.sync_copy(hrows0, osh.at[pl.ds(t * NPT + r * CHUNK, CHUNK)])
        return 0
    lax.fori_loop(0, NPT // CHUNK, _zcp, 0)
    plsc.subcore_barrier()

    pltpu.sync_copy(srcp.at[pl.ds(base, WG)], srcl)

    for b in range(2):
        pltpu.async_copy(h.at[srcl.at[0].at[b]], hrows[b], sem[b])

    def _group(jo, _):
        pltpu.sync_copy(dstp.at[base + jo], dstg)
        pltpu.sync_copy(attv.at[base + jo], attg)
        for ks in range(G8):
            b = ks % 2
            pltpu.make_async_copy(
                h.at[srcl.at[jo].at[ks]], hrows[b], sem[b]).wait()
            for k in range(CHUNK // 16):
                dsti[pl.ds(k * 16, 16)] = dstg[ks, pl.ds(k * 16, 16)]

            def _rgroup(r, _):
                av = attg[ks, pl.ds(r * 16, 16)]
                rbase = r * 16
                for i16 in range(16):
                    a = jnp.zeros((16,), _f32) + av[i16]
                    for k in range(D // 16):
                        hrows[b][rbase + i16, pl.ds(k * 16, 16)] = (
                            hrows[b][rbase + i16, pl.ds(k * 16, 16)] * a
                        )
                return 0
            lax.fori_loop(0, CHUNK // 16, _rgroup, 0)

            pltpu.sync_copy(hrows[b], osh.at[dsti], add=True)
            if ks < G8 - 2:
                pltpu.async_copy(
                    h.at[srcl.at[jo].at[ks + 2]], hrows[b], sem[b])
            else:
                @pl.when(jo + 1 < WG)
                def _():
                    pltpu.async_copy(
                        h.at[srcl.at[jo + 1].at[ks - 6]], hrows[b], sem[b])
        return 0
    lax.fori_loop(0, WG, _group, 0)
    plsc.subcore_barrier()

    @pl.when(c == 0)
    def _():
        def _cp(r, _):
            sl = pl.ds(t * NPT + r * CHUNK, CHUNK)
            pltpu.sync_copy(osh.at[sl], o0.at[sl])
            return 0
        lax.fori_loop(0, NPT // CHUNK, _cp, 0)

    @pl.when(c == 1)
    def _():
        def _cp(r, _):
            sl = pl.ds(t * NPT + r * CHUNK, CHUNK)
            pltpu.sync_copy(osh.at[sl], o1.at[sl])
            return 0
        lax.fori_loop(0, NPT // CHUNK, _cp, 0)


def kernel(x, e, W1, as1, ad1, b1, W2, as2, ad2, b2):
    xp = jnp.zeros((NP, D), _f32).at[:N].set(x)
    loop = jnp.arange(N, dtype=_i32)
    npad = EP - E - N
    pad = N + jnp.arange(npad, dtype=_i32) % (NP - N)
    srcp = jnp.concatenate([e[0], loop, pad]).reshape(EPG, G8, CHUNK)
    dstp = jnp.concatenate([e[1], loop, pad]).reshape(EPG, G8, CHUNK)
    a21 = jnp.stack([as1, ad1])
    a22 = jnp.stack([as2, ad2])
    b1r = b1.reshape(1, D)
    b2r = b2.reshape(1, D)

    h1, asv1, adv1 = _mm1(xp, W1, a21)
    att1 = _sc_b1(asv1.reshape(NP), adv1.reshape(NP), srcp, dstp)
    o1a, o1b = _sc_b2(h1, att1, srcp, dstp)

    h2, asv2, adv2 = _mm2(o1a, o1b, b1r, W2, a22)
    att2 = _sc_b1(asv2.reshape(NP), adv2.reshape(NP), srcp, dstp)
    o2a, o2b = _sc_b2(h2, att2, srcp, dstp)

    y = _final(o2a, o2b, b2r)
    return y[:N]


# speedup vs baseline: 28.2063x; 1.0454x over previous
import functools

import jax
import jax.numpy as jnp
from jax import lax
from jax.experimental import pallas as pl
from jax.experimental.pallas import tpu as pltpu
from jax.experimental.pallas import tpu_sc as plsc

N = 10000
E = 320000
D = 128

NC = 2
NS = 16
NW = NC * NS

CHUNK = 128
G8 = 8
NP = 10240
NPT = NP // NS
EPG = 352
EP = EPG * G8 * CHUNK
TG = EPG // NS
HG = TG // 2
WG = EPG // NW

_f32 = jnp.float32
_i32 = jnp.int32


BM = 512


def _mm1_body(x_ref, w_ref, a2_ref, h_ref, as_ref, ad_ref):
    hb = jnp.dot(x_ref[...], w_ref[...], preferred_element_type=_f32)
    h_ref[...] = hb
    av = lax.dot_general(a2_ref[...], hb, (((1,), (1,)), ((), ())))
    as_ref[...] = av[0:1]
    ad_ref[...] = av[1:2]


def _mm2_body(o0_ref, o1_ref, b_ref, w_ref, a2_ref, h_ref, as_ref, ad_ref):
    yb = o0_ref[...] + o1_ref[...] + b_ref[...]
    hb = jnp.dot(yb, w_ref[...], preferred_element_type=_f32)
    h_ref[...] = hb
    av = lax.dot_general(a2_ref[...], hb, (((1,), (1,)), ((), ())))
    as_ref[...] = av[0:1]
    ad_ref[...] = av[1:2]


def _final_body(o0_ref, o1_ref, b_ref, y_ref):
    y_ref[...] = jnp.maximum(o0_ref[...] + o1_ref[...] + b_ref[...], 0.0)


_row_spec = pl.BlockSpec((BM, D), lambda i: (i, 0))
_w_spec = pl.BlockSpec((D, D), lambda i: (0, 0))
_a2_spec = pl.BlockSpec((2, D), lambda i: (0, 0))
_b_spec = pl.BlockSpec((1, D), lambda i: (0, 0))
_av_spec = pl.BlockSpec((1, BM), lambda i: (0, i))

_mm_out = [
    jax.ShapeDtypeStruct((NP, D), _f32),
    jax.ShapeDtypeStruct((1, NP), _f32),
    jax.ShapeDtypeStruct((1, NP), _f32),
]

_mm1 = pl.pallas_call(
    _mm1_body,
    grid=(NP // BM,),
    in_specs=[_row_spec, _w_spec, _a2_spec],
    out_specs=[_row_spec, _av_spec, _av_spec],
    out_shape=_mm_out,
)

_mm2 = pl.pallas_call(
    _mm2_body,
    grid=(NP // BM,),
    in_specs=[_row_spec, _row_spec, _b_spec, _w_spec, _a2_spec],
    out_specs=[_row_spec, _av_spec, _av_spec],
    out_shape=_mm_out,
)

_final = pl.pallas_call(
    _final_body,
    grid=(NP // BM,),
    in_specs=[_row_spec, _row_spec, _b_spec],
    out_specs=_row_spec,
    out_shape=jax.ShapeDtypeStruct((NP, D), _f32),
)


_mesh = plsc.VectorSubcoreMesh(core_axis_name="c", subcore_axis_name="s")
_sc_params = pltpu.CompilerParams(use_tc_tiling_on_sc=False)


@functools.partial(
    pl.kernel,
    out_type=jax.ShapeDtypeStruct((EPG, G8, CHUNK), _f32),
    mesh=_mesh,
    scratch_types=[
        pltpu.VMEM((TG, G8, CHUNK), _i32),
        pltpu.VMEM((TG, G8, CHUNK), _i32),
        pltpu.VMEM((TG, G8, CHUNK), _f32),
        pltpu.VMEM((4, CHUNK), _f32),
        pltpu.VMEM((4, CHUNK), _f32),
        pltpu.VMEM((4, CHUNK), _f32),
        pltpu.VMEM((G8, CHUNK), _f32),
        pltpu.VMEM((NPT,), _f32),
        pltpu.VMEM_SHARED((NP,), _f32),
        pltpu.SemaphoreType.DMA,
        pltpu.SemaphoreType.DMA,
        pltpu.SemaphoreType.DMA,
        pltpu.SemaphoreType.DMA,
        pltpu.SemaphoreType.DMA,
    ],
    compiler_params=_sc_params,
)
def _sc_b1(asv, adv, srcp, dstp, attv,
           srcl, dstl, wfull, asg, adg, rd, attb, db,
           dsh, sem0, sem1, sem2, sem3, ssem):
    c = lax.axis_index("c")
    t = lax.axis_index("s")
    sem = (sem0, sem1, sem2, sem3)

    def _zero(i, _):
        db[pl.ds(i * 16, 16)] = jnp.zeros((16,), _f32)
        return 0
    lax.fori_loop(0, NPT // 16, _zero, 0)
    pltpu.sync_copy(db, dsh.at[pl.ds(t * NPT, NPT)])
    plsc.subcore_barrier()

    pltpu.sync_copy(srcp.at[pl.ds(t * TG, TG)], srcl)
    pltpu.sync_copy(dstp.at[pl.ds(t * TG, TG)], dstl)

    for b in range(4):
        pltpu.async_copy(asv.at[srcl.at[0].at[b]], asg.at[b], sem[b])
        pltpu.async_copy(adv.at[dstl.at[0].at[b]], adg.at[b], sem[b])

    def _group(jo, _):
        @pl.when(jo > 0)
        def _():
            for ks in range(G8):
                pltpu.make_async_copy(
                    wfull.at[jo - 1].at[ks],
                    dsh.at[dstl.at[jo - 1].at[ks]], ssem).wait()
        for ks in range(G8):
            b = ks % 4
            pltpu.make_async_copy(
                asv.at[srcl.at[jo].at[ks]], asg.at[b], sem[b]).wait()
            pltpu.make_async_copy(
                adv.at[dstl.at[jo].at[ks]], adg.at[b], sem[b]).wait()
            for k in range(CHUNK // 16):
                v = (asg[b, pl.ds(k * 16, 16)]
                     + adg[b, pl.ds(k * 16, 16)])
                v = jnp.where(v >= 0.0, v, 0.2 * v)
                wfull[jo, ks, pl.ds(k * 16, 16)] = jnp.exp(v)
            pltpu.async_copy(
                wfull.at[jo].at[ks], dsh.at[dstl.at[jo].at[ks]], ssem,
                add=True)
            if ks < G8 - 4:
                pltpu.async_copy(
                    asv.at[srcl.at[jo].at[ks + 4]], asg.at[b], sem[b])
                pltpu.async_copy(
                    adv.at[dstl.at[jo].at[ks + 4]], adg.at[b], sem[b])
            else:
                @pl.when(jo + 1 < TG)
                def _():
                    pltpu.async_copy(
                        asv.at[srcl.at[jo + 1].at[ks - 4]], asg.at[b],
                        sem[b])
                    pltpu.async_copy(
                        adv.at[dstl.at[jo + 1].at[ks - 4]], adg.at[b],
                        sem[b])
        return 0
    lax.fori_loop(0, TG, _group, 0)
    for ks in range(G8):
        pltpu.make_async_copy(
            wfull.at[TG - 1].at[ks],
            dsh.at[dstl.at[TG - 1].at[ks]], ssem).wait()
    plsc.subcore_barrier()

    pltpu.sync_copy(dsh.at[pl.ds(t * NPT, NPT)], db)

    def _recip(i, _):
        db[pl.ds(i * 16, 16)] = 1.0 / db[pl.ds(i * 16, 16)]
        return 0
    lax.fori_loop(0, NPT // 16, _recip, 0)
    pltpu.sync_copy(db, dsh.at[pl.ds(t * NPT, NPT)])
    plsc.subcore_barrier()

    g0 = c * HG
    for b in range(4):
        pltpu.async_copy(dsh.at[dstl.at[g0].at[b]], rd.at[b], sem[b])

    def _att(jo2, _):
        gsel = jo2 + c * HG
        for ks in range(G8):
            b = ks % 4
            pltpu.make_async_copy(
                dsh.at[dstl.at[gsel].at[ks]], rd.at[b], sem[b]).wait()
            for k in range(CHUNK // 16):
                attb[ks, pl.ds(k * 16, 16)] = (
                    wfull[gsel, ks, pl.ds(k * 16, 16)]
                    * rd[b, pl.ds(k * 16, 16)]
                )
            if ks < G8 - 4:
                pltpu.async_copy(
                    dsh.at[dstl.at[gsel].at[ks + 4]], rd.at[b], sem[b])
            else:
                @pl.when(jo2 + 1 < HG)
                def _():
                    pltpu.async_copy(
                        dsh.at[dstl.at[gsel + 1].at[ks - 4]], rd.at[b],
                        sem[b])
        pltpu.sync_copy(attb, attv.at[t * TG + gsel])
        return 0
    lax.fori_loop(0, HG, _att, 0)


@functools.partial(
    pl.kernel,
    out_type=[
        jax.ShapeDtypeStruct((NP, D), _f32),
        jax.ShapeDtypeStruct((NP, D), _f32),
    ],
    mesh=_mesh,
    scratch_types=[
        pltpu.VMEM((WG, G8, CHUNK), _i32),
        pltpu.VMEM((G8, CHUNK), _i32),
        pltpu.VMEM((G8, CHUNK), _f32),
        pltpu.VMEM((CHUNK, D), _f32),
        pltpu.VMEM((CHUNK, D), _f32),
        pltpu.VMEM_SHARED((NP, D), _f32),
        pltpu.SemaphoreType.DMA,
        pltpu.SemaphoreType.DMA,
    ],
    compiler_params=_sc_params,
)
def _sc_b2(h, attv, srcp, dstp, o0, o1,
           srcl, dstg, attg, hrows0, hrows1, osh, sem0, sem1):
    c = lax.axis_index("c")
    t = lax.axis_index("s")
    wid = c * NS + t
    base = t * TG + c * HG
    hrows = (hrows0, hrows1)
    sem = (sem0, sem1)

    def _zrow(i, _):
        for k in range(D // 16):
            hrows0[i, pl.ds(k * 16, 16)] = jnp.zeros((16,), _f32)
        return 0
    lax.fori_loop(0, CHUNK, _zrow, 0)

    def _zcp(r, _):
        pltpu.sync_copy(hrows0, osh.at[pl.ds(t * NPT + r * CHUNK, CHUNK)])
        return 0
    lax.fori_loop(0, NPT // CHUNK, _zcp, 0)
    plsc.subcore_barrier()

    pltpu.sync_copy(srcp.at[pl.ds(base, WG)], srcl)

    for b in range(2):
        pltpu.async_copy(h.at[srcl.at[0].at[b]], hrows[b], sem[b])

    def _group(jo, _):
        pltpu.sync_copy(dstp.at[base + jo], dstg)
        pltpu.sync_copy(attv.at[base + jo], attg)
        for ks in range(G8):
            b = ks % 2
            pltpu.make_async_copy(
                h.at[srcl.at[jo].at[ks]], hrows[b], sem[b]).wait()

            def _rgroup(r, _):
                av = attg[ks, pl.ds(r * 16, 16)]
                rbase = r * 16
                for i16 in range(16):
                    a = jnp.zeros((16,), _f32) + av[i16]
                    for k in range(D // 16):
                        hrows[b][rbase + i16, pl.ds(k * 16, 16)] = (
                            hrows[b][rbase + i16, pl.ds(k * 16, 16)] * a
                        )
                return 0
            lax.fori_loop(0, CHUNK // 16, _rgroup, 0)

            pltpu.sync_copy(hrows[b], osh.at[dstg.at[ks]], add=True)
            if ks < G8 - 2:
                pltpu.async_copy(
                    h.at[srcl.at[jo].at[ks + 2]], hrows[b], sem[b])
            else:
                @pl.when(jo + 1 < WG)
                def _():
                    pltpu.async_copy(
                        h.at[srcl.at[jo + 1].at[ks - 6]], hrows[b], sem[b])
        return 0
    lax.fori_loop(0, WG, _group, 0)
    plsc.subcore_barrier()

    @pl.when(c == 0)
    def _():
        def _cp(r, _):
            sl = pl.ds(t * NPT + r * CHUNK, CHUNK)
            pltpu.sync_copy(osh.at[sl], o0.at[sl])
            return 0
        lax.fori_loop(0, NPT // CHUNK, _cp, 0)

    @pl.when(c == 1)
    def _():
        def _cp(r, _):
            sl = pl.ds(t * NPT + r * CHUNK, CHUNK)
            pltpu.sync_copy(osh.at[sl], o1.at[sl])
            return 0
        lax.fori_loop(0, NPT // CHUNK, _cp, 0)


def kernel(x, e, W1, as1, ad1, b1, W2, as2, ad2, b2):
    xp = jnp.zeros((NP, D), _f32).at[:N].set(x)
    loop = jnp.arange(N, dtype=_i32)
    npad = EP - E - N
    pad = N + jnp.arange(npad, dtype=_i32) % (NP - N)
    srcp = jnp.concatenate([e[0], loop, pad]).reshape(EPG, G8, CHUNK)
    dstp = jnp.concatenate([e[1], loop, pad]).reshape(EPG, G8, CHUNK)
    a21 = jnp.stack([as1, ad1])
    a22 = jnp.stack([as2, ad2])
    b1r = b1.reshape(1, D)
    b2r = b2.reshape(1, D)

    h1, asv1, adv1 = _mm1(xp, W1, a21)
    att1 = _sc_b1(asv1.reshape(NP), adv1.reshape(NP), srcp, dstp)
    o1a, o1b = _sc_b2(h1, att1, srcp, dstp)

    h2, asv2, adv2 = _mm2(o1a, o1b, b1r, W2, a22)
    att2 = _sc_b1(asv2.reshape(NP), adv2.reshape(NP), srcp, dstp)
    o2a, o2b = _sc_b2(h2, att2, srcp, dstp)

    y = _final(o2a, o2b, b2r)
    return y[:N]
